# baseline (speedup 1.0000x reference)
import functools

import jax
import jax.numpy as jnp
from jax import lax
from jax.experimental import pallas as pl
from jax.experimental.pallas import tpu as pltpu
from jax.experimental.pallas import tpu_sc as plsc

T = 3; N = 10000; E = 160000; D = 128; G = 128; H = 128; MH = 64
NC = 2; NS = 16; NW = NC * NS; L = 16
EPW = 5120
EP = EPW * NW
CH = 128
NCH = EPW // CH
NACC = 10240
NRD = 80
ACH = 64
ANCH = EPW // ACH
NBUF = 2
NACCA = 10112

_mesh = plsc.VectorSubcoreMesh(core_axis_name="c", subcore_axis_name="s")
_sc_params = pltpu.CompilerParams(needs_layout_passes=False)


def _worker_id():
    return lax.axis_index("s") * NC + lax.axis_index("c")


@functools.partial(
    pl.kernel,
    out_type=(
        jax.ShapeDtypeStruct((EP,), jnp.float32),
        jax.ShapeDtypeStruct((NC, 2 * NRD, 128), jnp.float32),
    ),
    mesh=_mesh,
    compiler_params=_sc_params,
    scratch_types=(
        pltpu.VMEM((EPW,), jnp.int32),
        pltpu.VMEM((NCH, CH), jnp.int32),
        pltpu.VMEM((CH, 128), jnp.float32),
        pltpu.VMEM((CH, 128), jnp.float32),
        pltpu.VMEM((CH, 128), jnp.float32),
        pltpu.VMEM((CH, 128), jnp.float32),
        pltpu.VMEM((8, 128), jnp.float32),
        pltpu.VMEM((CH,), jnp.float32),
        pltpu.VMEM((EPW,), jnp.float32),
        pltpu.VMEM((2 * NRD, 128), jnp.float32),
        pltpu.VMEM((NRD,), jnp.int32),
        pltpu.VMEM((NRD,), jnp.int32),
        pltpu.VMEM_SHARED((2 * NRD, 128), jnp.float32),
        pltpu.SemaphoreType.DMA,
        pltpu.SemaphoreType.DMA,
        pltpu.SemaphoreType.DMA,
        pltpu.SemaphoreType.DMA,
    ),
)
def _em_kernel(ab_hbm, src_hbm, dst_hbm, consts_hbm, iota_hbm,
               em_hbm, degs_hbm,
               src_v, dst_v, a0, a1, b0, b1, consts_v, empre, em_buf, dp,
               idxa, idxb, spdeg, sa0, sa1, sb0, sb1):
    c = lax.axis_index("c")
    s = lax.axis_index("s")
    w = s * NC + c
    pltpu.sync_copy(consts_hbm, consts_v)
    pltpu.sync_copy(src_hbm.at[pl.ds(w * EPW, EPW)], src_v)
    pltpu.sync_copy(dst_hbm.at[w], dst_v)
    pltpu.sync_copy(iota_hbm.at[pl.ds(0, NRD)], idxa)
    pltpu.sync_copy(iota_hbm.at[pl.ds(NRD, NRD)], idxb)

    zv = jnp.zeros((L,), jnp.float32)

    def _zero(i, carry):
        dp[i // 8, pl.ds((i % 8) * L, L)] = zv
        return carry

    lax.fori_loop(0, 2 * NRD * 8, _zero, 0)

    @pl.when(s == 0)
    def _():
        pltpu.sync_copy(dp, spdeg)

    plsc.subcore_barrier()

    cvec = [consts_v[0, pl.ds(k * L, L)] for k in range(4)]
    wvec = [consts_v[1, pl.ds(k * L, L)] for k in range(4)]
    be2v = consts_v[2, pl.ds(0, L)]
    lane = lax.iota(jnp.int32, 16)
    mask15 = lane == 15
    ones16 = jnp.ones((L,), jnp.float32)

    bufs = ((a0, sa0, b0, sb0), (a1, sa1, b1, sb1))
    dummy = ab_hbm.at[pl.ds(0, CH)]
    for bi in range(2):
        arows, sa, brows, sb = bufs[bi]
        pltpu.async_copy(ab_hbm.at[src_v.at[pl.ds(bi * CH, CH)]], arows, sa)
        pltpu.async_copy(ab_hbm.at[dst_v.at[bi]], brows, sb)

    def _group(g0, carry):
        for bi in range(2):
            j = g0 * 2 + bi
            arows, sa, brows, sb = bufs[bi]
            pltpu.make_async_copy(dummy, arows, sa).wait()
            pltpu.make_async_copy(dummy, brows, sb).wait()

            def _edge(e, ecarry):
                acc = None
                for k in range(4):
                    av = arows[e, pl.ds(k * L, L)]
                    bv = brows[e, pl.ds(MH + k * L, L)]
                    g = jnp.maximum(av + bv + cvec[k], 0.0) * wvec[k]
                    acc = g if acc is None else acc + g
                cs = plsc.cumsum(acc)
                plsc.store_scatter(empre, [jnp.full((L,), e, jnp.int32)], cs,
                                   mask=mask15)
                return ecarry

            lax.fori_loop(0, CH, _edge, 0, unroll=4)

            def _grp(g, gcarry):
                sv = empre[pl.ds(g * L, L)] + be2v
                emv = 1.0 / (1.0 + jnp.exp(-sv))
                em_buf[pl.ds(j * CH + g * L, L)] = emv
                dstv = dst_v[j, pl.ds(g * L, L)]
                row = lax.shift_right_logical(dstv, 7)
                col = lax.bitwise_and(dstv, 127)
                plsc.addupdate_scatter(dp, [row, col], ones16)
                plsc.addupdate_scatter(dp, [row + NRD, col], emv)
                return gcarry

            lax.fori_loop(0, 8, _grp, 0, unroll=2)

            @pl.when(j + 2 < NCH)
            def _():
                pltpu.async_copy(
                    ab_hbm.at[src_v.at[pl.ds((j + 2) * CH, CH)]], arows, sa)
                pltpu.async_copy(ab_hbm.at[dst_v.at[j + 2]], brows, sb)
        return carry

    lax.fori_loop(0, NCH // 2, _group, 0)

    pltpu.sync_copy(em_buf, em_hbm.at[pl.ds(w * EPW, EPW)])
    pltpu.sync_copy(dp.at[pl.ds(0, NRD)], spdeg.at[idxa], add=True)
    pltpu.sync_copy(dp.at[pl.ds(NRD, NRD)], spdeg.at[idxb], add=True)
    plsc.subcore_barrier()

    @pl.when(s == 0)
    def _():
        pltpu.sync_copy(spdeg, degs_hbm.at[c])


NB = 2


def _agg_body(scaled, args):
    if scaled:
        (table_hbm, src_hbm, dst_hbm, scl_hbm, parts_hbm,
         src_v, dst_v, scl_v, rows2, zb, acc, sg, ss) = args
    else:
        (table_hbm, src_hbm, dst_hbm, parts_hbm,
         src_v, dst_v, rows2, zb, acc, sg, ss) = args
    c = lax.axis_index("c")
    s = lax.axis_index("s")
    w = s * NC + c
    pltpu.sync_copy(src_hbm.at[pl.ds(w * EPW, EPW)], src_v)
    pltpu.sync_copy(dst_hbm.at[w], dst_v)
    if scaled:
        pltpu.sync_copy(scl_hbm.at[pl.ds(w * EPW, EPW)], scl_v)

    zv = jnp.zeros((L,), jnp.float32)

    def _zero(i, carry):
        zb[i // 8, pl.ds((i % 8) * L, L)] = zv
        return carry

    lax.fori_loop(0, 79 * 8, _zero, 0)

    def _zacc(k, carry):
        pltpu.sync_copy(zb, acc.at[pl.ds(s * (NACCA // NS) + k * 79, 79)])
        return carry

    lax.fori_loop(0, NACCA // NS // 79, _zacc, 0)
    plsc.subcore_barrier()

    dummy = table_hbm.at[pl.ds(0, ACH)]
    slots = tuple(rows2.at[pl.ds(bi * ACH, ACH)] for bi in range(NBUF))
    for bi in range(2):
        pltpu.async_copy(table_hbm.at[src_v.at[pl.ds(bi * ACH, ACH)]],
                         slots[bi], sg)

    def _chunk(j, carry):
        nxt = jnp.minimum(j + NBUF, ANCH - 1)
        pltpu.make_async_copy(dummy, slots[0], sg).wait()
        for bi in range(NBUF):
            @pl.when(j % NBUF == bi)
            def _():
                rows = slots[bi]
                if scaled:
                    def _edge(e, ecarry):
                        sv = plsc.load_gather(
                            scl_v, [jnp.full((L,), j * ACH + e, jnp.int32)])
                        for k in range(8):
                            rows[e, pl.ds(k * L, L)] = (
                                rows[e, pl.ds(k * L, L)] * sv)
                        return ecarry

                    lax.fori_loop(0, ACH, _edge, 0, unroll=2)
                pltpu.sync_copy(rows, acc.at[dst_v.at[j]], add=True)
                pltpu.async_copy(table_hbm.at[src_v.at[pl.ds(nxt * ACH,
                                                             ACH)]],
                                 rows, sg)
        return carry

    lax.fori_loop(0, ANCH, _chunk, 0)
    for _ in range(NBUF):
        pltpu.make_async_copy(dummy, slots[0], sg).wait()
    plsc.subcore_barrier()
    rpt = NACCA // NS
    pltpu.sync_copy(acc.at[pl.ds(s * rpt, rpt)],
                    parts_hbm.at[c].at[pl.ds(s * rpt, rpt)])


_agg_out = jax.ShapeDtypeStruct((NC, NACCA, 128), jnp.float32)
_ring = (pltpu.VMEM((NBUF * ACH, 128), jnp.float32),)
_sems = (pltpu.SemaphoreType.DMA, pltpu.SemaphoreType.DMA)
_agg_scratch = (
    pltpu.VMEM((EPW,), jnp.int32),
    pltpu.VMEM((ANCH, ACH), jnp.int32),
) + _ring + (
    pltpu.VMEM((79, 128), jnp.float32),
    pltpu.VMEM_SHARED((NACCA, 128), jnp.float32),
) + _sems
_agg_scale_scratch = (
    pltpu.VMEM((EPW,), jnp.int32),
    pltpu.VMEM((ANCH, ACH), jnp.int32),
    pltpu.VMEM((EPW,), jnp.float32),
) + _ring + (
    pltpu.VMEM((79, 128), jnp.float32),
    pltpu.VMEM_SHARED((NACCA, 128), jnp.float32),
) + _sems


@functools.partial(pl.kernel, out_type=_agg_out, mesh=_mesh,
                   compiler_params=_sc_params, scratch_types=_agg_scratch)
def _agg_kernel(*args):
    _agg_body(False, args)


@functools.partial(pl.kernel, out_type=_agg_out, mesh=_mesh,
                   compiler_params=_sc_params, scratch_types=_agg_scale_scratch)
def _agg_scale_kernel(*args):
    _agg_body(True, args)


BR = 1000


def _pre_body(x_ref, wm1_ref, wm2_ref, cm_ref, bm2_ref, weab_ref,
              nm_ref, xm_ref, ab_ref):
    x = x_ref[...]
    hmid = jnp.maximum(
        jnp.dot(x, wm1_ref[...], preferred_element_type=jnp.float32)
        + cm_ref[...], 0.0)
    nm = jax.nn.sigmoid(
        jnp.dot(hmid, wm2_ref[...], preferred_element_type=jnp.float32)
        + bm2_ref[...])
    nm_ref[...] = nm
    xm_ref[...] = x * nm
    ab_ref[...] = jnp.dot(x, weab_ref[...],
                          preferred_element_type=jnp.float32)


def _pre_call(x, wm1a, wm2, cm, bm2, weab):
    grid = (N // BR,)
    full = lambda shape: pl.BlockSpec(shape, lambda i: (0, 0))
    return pl.pallas_call(
        _pre_body,
        grid=grid,
        in_specs=[
            pl.BlockSpec((BR, D), lambda i: (i, 0)),
            full((D, MH)), full((MH, 1)), full((1, MH)), full((1, 1)),
            full((D, 2 * MH)),
        ],
        out_specs=[
            pl.BlockSpec((BR, 1), lambda i: (i, 0)),
            pl.BlockSpec((BR, D), lambda i: (i, 0)),
            pl.BlockSpec((BR, 2 * MH), lambda i: (i, 0)),
        ],
        out_shape=[
            jax.ShapeDtypeStruct((N, 1), jnp.float32),
            jax.ShapeDtypeStruct((N, D), jnp.float32),
            jax.ShapeDtypeStruct((N, 2 * MH), jnp.float32),
        ],
    )(x, wm1a, wm2, cm, bm2, weab)


def _conv_body(inp_ref, p0_ref, p1_ref, dinv_ref, ws_ref, bs_ref, wn_ref,
               bn_ref, out_ref):
    agg = (p0_ref[...] + p1_ref[...]) * dinv_ref[...]
    out_ref[...] = jnp.maximum(
        jnp.dot(inp_ref[...], ws_ref[...], preferred_element_type=jnp.float32)
        + bs_ref[...]
        + jnp.dot(agg, wn_ref[...], preferred_element_type=jnp.float32)
        + bn_ref[...], 0.0)


def _conv_call(inp, p0, p1, dinv, ws, bs, wn, bn):
    grid = (N // BR,)
    blk = pl.BlockSpec((BR, H), lambda i: (i, 0))
    full = lambda shape: pl.BlockSpec(shape, lambda i: (0, 0))
    return pl.pallas_call(
        _conv_body,
        grid=grid,
        in_specs=[blk, blk, blk, pl.BlockSpec((BR, 1), lambda i: (i, 0)),
                  full((H, H)), full((1, H)), full((H, H)), full((1, H))],
        out_specs=blk,
        out_shape=jax.ShapeDtypeStruct((N, H), jnp.float32),
    )(inp, p0, p1, dinv, ws, bs, wn, bn)


def _post_body(p0_ref, p1_ref, dinv_ref, h2_ref, sp_ref, pooled_ref):
    i = pl.program_id(0)
    sp = (p0_ref[...] + p1_ref[...]) * dinv_ref[...]
    sp_ref[...] = sp
    contrib = jnp.sum(h2_ref[...] * sp, axis=0, keepdims=True) * (1.0 / N)

    @pl.when(i == 0)
    def _():
        pooled_ref[...] = contrib

    @pl.when(i > 0)
    def _():
        pooled_ref[...] = pooled_ref[...] + contrib


def _post_call(p0, p1, dinv, h2):
    grid = (N // BR,)
    blk = pl.BlockSpec((BR, H), lambda i: (i, 0))
    return pl.pallas_call(
        _post_body,
        grid=grid,
        in_specs=[blk, blk, pl.BlockSpec((BR, 1), lambda i: (i, 0)), blk],
        out_specs=[blk, pl.BlockSpec((1, H), lambda i: (0, 0))],
        out_shape=[jax.ShapeDtypeStruct((N, H), jnp.float32),
                   jax.ShapeDtypeStruct((1, H), jnp.float32)],
    )(p0, p1, dinv, h2)


def _head_body(pooled_ref, gf_ref, wg_ref, bg_ref, wih_ref, bih_ref, whh_ref,
               bhh_ref, wa_ref, ba_ref, wt_ref, bt_ref, wr1_ref, br1_ref,
               wr2_ref, br2_ref, pred_ref, wts_ref, tmask_ref):
    gs = jnp.maximum(
        jnp.dot(gf_ref[...], wg_ref[...], preferred_element_type=jnp.float32)
        + bg_ref[...], 0.0)
    seq = jnp.concatenate([pooled_ref[...], gs], axis=1)
    gi = jnp.dot(seq, wih_ref[...], preferred_element_type=jnp.float32) \
        + bih_ref[...]
    h = jnp.zeros((1, H), jnp.float32)
    outs = []
    for t in range(T):
        gh = jnp.dot(h, whh_ref[...], preferred_element_type=jnp.float32) \
            + bhh_ref[...]
        git = gi[t:t + 1]
        r = jax.nn.sigmoid(git[:, :H] + gh[:, :H])
        z = jax.nn.sigmoid(git[:, H:2 * H] + gh[:, H:2 * H])
        n = jnp.tanh(git[:, 2 * H:] + r * gh[:, 2 * H:])
        h = (1.0 - z) * n + z * h
        outs.append(h)
    gru = jnp.concatenate(outs, axis=0)
    tm = jax.nn.sigmoid(jnp.dot(gru, wt_ref[...],
                                preferred_element_type=jnp.float32)
                        + bt_ref[...])
    sc = jnp.dot(gru, wa_ref[...], preferred_element_type=jnp.float32) \
        + ba_ref[...]
    m = jnp.max(sc)
    ex = jnp.exp(sc - m)
    wts = ex / jnp.sum(ex)
    tmask_ref[...] = tm
    wts_ref[...] = wts
    ctx = jnp.sum(gru * wts * tm, axis=0, keepdims=True)
    pred_ref[...] = jnp.dot(
        jnp.maximum(jnp.dot(ctx, wr1_ref[...],
                            preferred_element_type=jnp.float32)
                    + br1_ref[...], 0.0),
        wr2_ref[...], preferred_element_type=jnp.float32) + br2_ref[...]


def _head_call(pooled3, gf_all, p):
    args = (pooled3, gf_all, p['Wg'], p['bg'].reshape(1, H),
            p['Wih'], p['bih'].reshape(1, 3 * H),
            p['Whh'], p['bhh'].reshape(1, 3 * H),
            p['Wa'], p['ba'].reshape(1, 1), p['Wt'], p['bt'].reshape(1, 1),
            p['Wr1'], p['br1'].reshape(1, H), p['Wr2'],
            p['br2'].reshape(1, 1))
    return pl.pallas_call(
        _head_body,
        out_shape=[jax.ShapeDtypeStruct((1, 1), jnp.float32),
                   jax.ShapeDtypeStruct((T, 1), jnp.float32),
                   jax.ShapeDtypeStruct((T, 1), jnp.float32)],
    )(*args)


def kernel(x_all, edge_index_all, gf_all, params):
    p = params
    weab = jnp.concatenate([p['We1'][:D], p['We1'][D:2 * D]], axis=1)
    wm1a = p['Wm1'][:D]
    iota2 = jnp.arange(2 * NRD, dtype=jnp.int32)
    pad_i = jnp.zeros((EP - E,), jnp.int32)
    pad_d = jnp.full((EP - E,), N, jnp.int32)

    sp_list, em_list, nm_list, pooled_list = [], [], [], []
    for t in range(T):
        tf = float(t) / float(T)
        x = x_all[t]
        src_pad = jnp.concatenate([edge_index_all[t, 0], pad_i])
        dst_pad = jnp.concatenate([edge_index_all[t, 1], pad_d])
        dst2d = dst_pad.reshape(NW, NCH, CH)
        dst2da = dst_pad.reshape(NW, ANCH, ACH)

        ce = tf * p['We1'][2 * D] + p['be1']
        cm = tf * p['Wm1'][D] + p['bm1']
        consts = jnp.zeros((8, 128), jnp.float32)
        consts = consts.at[0, :MH].set(ce)
        consts = consts.at[1, :MH].set(p['We2'][:, 0])
        consts = consts.at[2].set(p['be2'][0])

        nm2d, xm, AB = _pre_call(x, wm1a, p['Wm2'], cm.reshape(1, MH),
                                 p['bm2'].reshape(1, 1), weab)
        ABp = jnp.concatenate([AB, jnp.zeros((8, 2 * MH), jnp.float32)])
        em_pad, degs = _em_kernel(ABp, src_pad, dst2d, consts, iota2)

        degsum = degs[0] + degs[1]
        deg_u = jnp.maximum(degsum[:NRD].reshape(-1)[:N], 1.0)
        deg_e = jnp.maximum(degsum[NRD:].reshape(-1)[:N], 1.0)
        dinv_u = (1.0 / deg_u).reshape(N, 1)
        dinv_e = (1.0 / deg_e).reshape(N, 1)

        parts1 = _agg_kernel(xm, src_pad, dst2da)
        h1 = _conv_call(xm, parts1[0], parts1[1], dinv_u, p['Ws1'],
                        p['bs1'].reshape(1, H), p['Wn1'],
                        p['bn1'].reshape(1, H))
        parts2 = _agg_kernel(h1, src_pad, dst2da)
        h2 = _conv_call(h1, parts2[0], parts2[1], dinv_u, p['Ws2'],
                        p['bs2'].reshape(1, H), p['Wn2'],
                        p['bn2'].reshape(1, H))
        parts3 = _agg_scale_kernel(h2, src_pad, dst2da, em_pad)
        sp, pooled = _post_call(parts3[0], parts3[1], dinv_e, h2)

        sp_list.append(sp)
        em_list.append(em_pad[:E])
        nm_list.append(nm2d[:, 0])
        pooled_list.append(pooled)

    pooled3 = jnp.concatenate(pooled_list, axis=0)
    pred, wts, tmask = _head_call(pooled3, gf_all, p)
    return (pred[0, 0], wts[:, 0], tmask[:, 0], jnp.stack(sp_list),
            jnp.stack(em_list), jnp.stack(nm_list))


# speedup vs baseline: 1.0003x; 1.0003x over previous
import functools

import jax
import jax.numpy as jnp
from jax import lax
from jax.experimental import pallas as pl
from jax.experimental.pallas import tpu as pltpu
from jax.experimental.pallas import tpu_sc as plsc

T = 3; N = 10000; E = 160000; D = 128; G = 128; H = 128; MH = 64
NC = 2; NS = 16; NW = NC * NS; L = 16
EPW = 5120
EP = EPW * NW
CH = 128
NCH = EPW // CH
NACC = 10240
NRD = 80
ACH = 64
ANCH = EPW // ACH
NBUF = 2
NACCA = 10112

_mesh = plsc.VectorSubcoreMesh(core_axis_name="c", subcore_axis_name="s")
_sc_params = pltpu.CompilerParams(needs_layout_passes=False)


@functools.partial(
    pl.kernel,
    out_type=(
        jax.ShapeDtypeStruct((EP,), jnp.float32),
        jax.ShapeDtypeStruct((NC, 2 * NRD, 128), jnp.float32),
    ),
    mesh=_mesh,
    compiler_params=_sc_params,
    scratch_types=(
        pltpu.VMEM((EPW,), jnp.int32),
        pltpu.VMEM((NCH, CH), jnp.int32),
        pltpu.VMEM((CH, 128), jnp.float32),
        pltpu.VMEM((CH, 128), jnp.float32),
        pltpu.VMEM((CH, 128), jnp.float32),
        pltpu.VMEM((CH, 128), jnp.float32),
        pltpu.VMEM((8, 128), jnp.float32),
        pltpu.VMEM((CH,), jnp.float32),
        pltpu.VMEM((EPW,), jnp.float32),
        pltpu.VMEM((2 * NRD, 128), jnp.float32),
        pltpu.VMEM((NRD,), jnp.int32),
        pltpu.VMEM((NRD,), jnp.int32),
        pltpu.VMEM_SHARED((2 * NRD, 128), jnp.float32),
        pltpu.SemaphoreType.DMA,
        pltpu.SemaphoreType.DMA,
        pltpu.SemaphoreType.DMA,
        pltpu.SemaphoreType.DMA,
    ),
)
def _em_kernel(ab_hbm, src_hbm, dst_hbm, consts_hbm, iota_hbm,
               em_hbm, degs_hbm,
               src_v, dst_v, a0, a1, b0, b1, consts_v, empre, em_buf, dp,
               idxa, idxb, spdeg, sa0, sa1, sb0, sb1):
    c = lax.axis_index("c")
    s = lax.axis_index("s")
    w = s * NC + c
    pltpu.sync_copy(consts_hbm, consts_v)
    pltpu.sync_copy(src_hbm.at[pl.ds(w * EPW, EPW)], src_v)
    pltpu.sync_copy(dst_hbm.at[w], dst_v)
    pltpu.sync_copy(iota_hbm.at[pl.ds(0, NRD)], idxa)
    pltpu.sync_copy(iota_hbm.at[pl.ds(NRD, NRD)], idxb)

    zv = jnp.zeros((L,), jnp.float32)

    def _zero(i, carry):
        dp[i // 8, pl.ds((i % 8) * L, L)] = zv
        return carry

    lax.fori_loop(0, 2 * NRD * 8, _zero, 0)

    @pl.when(s == 0)
    def _():
        pltpu.sync_copy(dp, spdeg)

    plsc.subcore_barrier()

    cvec = [consts_v[0, pl.ds(k * L, L)] for k in range(4)]
    wvec = [consts_v[1, pl.ds(k * L, L)] for k in range(4)]
    be2v = consts_v[2, pl.ds(0, L)]
    lane = lax.iota(jnp.int32, 16)
    mask15 = lane == 15
    ones16 = jnp.ones((L,), jnp.float32)

    bufs = ((a0, sa0, b0, sb0), (a1, sa1, b1, sb1))
    dummy = ab_hbm.at[pl.ds(0, CH)]
    for bi in range(2):
        arows, sa, brows, sb = bufs[bi]
        pltpu.async_copy(ab_hbm.at[src_v.at[pl.ds(bi * CH, CH)]], arows, sa)
        pltpu.async_copy(ab_hbm.at[dst_v.at[bi]], brows, sb)

    def _group(g0, carry):
        for bi in range(2):
            j = g0 * 2 + bi
            arows, sa, brows, sb = bufs[bi]
            pltpu.make_async_copy(dummy, arows, sa).wait()
            pltpu.make_async_copy(dummy, brows, sb).wait()

            def _edge(e, ecarry):
                acc = None
                for k in range(4):
                    av = arows[e, pl.ds(k * L, L)]
                    bv = brows[e, pl.ds(MH + k * L, L)]
                    g = jnp.maximum(av + bv + cvec[k], 0.0) * wvec[k]
                    acc = g if acc is None else acc + g
                cs = plsc.cumsum(acc)
                plsc.store_scatter(empre, [jnp.full((L,), e, jnp.int32)], cs,
                                   mask=mask15)
                return ecarry

            lax.fori_loop(0, CH, _edge, 0, unroll=4)

            def _grp(g, gcarry):
                sv = empre[pl.ds(g * L, L)] + be2v
                emv = 1.0 / (1.0 + jnp.exp(-sv))
                em_buf[pl.ds(j * CH + g * L, L)] = emv
                dstv = dst_v[j, pl.ds(g * L, L)]
                row = lax.shift_right_logical(dstv, 7)
                col = lax.bitwise_and(dstv, 127)
                plsc.addupdate_scatter(dp, [row, col], ones16)
                plsc.addupdate_scatter(dp, [row + NRD, col], emv)
                return gcarry

            lax.fori_loop(0, 8, _grp, 0, unroll=2)

            @pl.when(j + 2 < NCH)
            def _():
                pltpu.async_copy(
                    ab_hbm.at[src_v.at[pl.ds((j + 2) * CH, CH)]], arows, sa)
                pltpu.async_copy(ab_hbm.at[dst_v.at[j + 2]], brows, sb)
        return carry

    lax.fori_loop(0, NCH // 2, _group, 0)

    pltpu.sync_copy(em_buf, em_hbm.at[pl.ds(w * EPW, EPW)])
    pltpu.sync_copy(dp.at[pl.ds(0, NRD)], spdeg.at[idxa], add=True)
    pltpu.sync_copy(dp.at[pl.ds(NRD, NRD)], spdeg.at[idxb], add=True)
    plsc.subcore_barrier()

    @pl.when(s == 0)
    def _():
        pltpu.sync_copy(spdeg, degs_hbm.at[c])


def _agg_body(scaled, args):
    if scaled:
        (table_hbm, src_hbm, dst_hbm, scl_hbm, parts_hbm,
         src_v, dst_v, scl_v, rows2, zb, acc, sg, ss) = args
    else:
        (table_hbm, src_hbm, dst_hbm, parts_hbm,
         src_v, dst_v, rows2, zb, acc, sg, ss) = args
    c = lax.axis_index("c")
    s = lax.axis_index("s")
    w = s * NC + c
    pltpu.sync_copy(src_hbm.at[pl.ds(w * EPW, EPW)], src_v)
    pltpu.sync_copy(dst_hbm.at[w], dst_v)
    if scaled:
        pltpu.sync_copy(scl_hbm.at[pl.ds(w * EPW, EPW)], scl_v)

    zv = jnp.zeros((L,), jnp.float32)

    def _zero(i, carry):
        zb[i // 8, pl.ds((i % 8) * L, L)] = zv
        return carry

    lax.fori_loop(0, 79 * 8, _zero, 0)

    def _zacc(k, carry):
        pltpu.sync_copy(zb, acc.at[pl.ds(s * (NACCA // NS) + k * 79, 79)])
        return carry

    lax.fori_loop(0, NACCA // NS // 79, _zacc, 0)
    plsc.subcore_barrier()

    dummy = table_hbm.at[pl.ds(0, ACH)]
    slots = tuple(rows2.at[pl.ds(bi * ACH, ACH)] for bi in range(NBUF))
    for bi in range(2):
        pltpu.async_copy(table_hbm.at[src_v.at[pl.ds(bi * ACH, ACH)]],
                         slots[bi], sg)

    def _chunk(j, carry):
        nxt = jnp.minimum(j + NBUF, ANCH - 1)
        pltpu.make_async_copy(dummy, slots[0], sg).wait()
        for bi in range(NBUF):
            @pl.when(j % NBUF == bi)
            def _():
                rows = slots[bi]
                if scaled:
                    def _edge(e, ecarry):
                        sv = plsc.load_gather(
                            scl_v, [jnp.full((L,), j * ACH + e, jnp.int32)])
                        for k in range(8):
                            rows[e, pl.ds(k * L, L)] = (
                                rows[e, pl.ds(k * L, L)] * sv)
                        return ecarry

                    lax.fori_loop(0, ACH, _edge, 0, unroll=2)
                pltpu.sync_copy(rows, acc.at[dst_v.at[j]], add=True)
                pltpu.async_copy(table_hbm.at[src_v.at[pl.ds(nxt * ACH,
                                                             ACH)]],
                                 rows, sg)
        return carry

    lax.fori_loop(0, ANCH, _chunk, 0)
    for _ in range(NBUF):
        pltpu.make_async_copy(dummy, slots[0], sg).wait()
    plsc.subcore_barrier()
    rpt = NACCA // NS
    pltpu.sync_copy(acc.at[pl.ds(s * rpt, rpt)],
                    parts_hbm.at[c].at[pl.ds(s * rpt, rpt)])


_agg_out = jax.ShapeDtypeStruct((NC, NACCA, 128), jnp.float32)
_ring = (pltpu.VMEM((NBUF * ACH, 128), jnp.float32),)
_sems = (pltpu.SemaphoreType.DMA, pltpu.SemaphoreType.DMA)
_agg_scratch = (
    pltpu.VMEM((EPW,), jnp.int32),
    pltpu.VMEM((ANCH, ACH), jnp.int32),
) + _ring + (
    pltpu.VMEM((79, 128), jnp.float32),
    pltpu.VMEM_SHARED((NACCA, 128), jnp.float32),
) + _sems
_agg_scale_scratch = (
    pltpu.VMEM((EPW,), jnp.int32),
    pltpu.VMEM((ANCH, ACH), jnp.int32),
    pltpu.VMEM((EPW,), jnp.float32),
) + _ring + (
    pltpu.VMEM((79, 128), jnp.float32),
    pltpu.VMEM_SHARED((NACCA, 128), jnp.float32),
) + _sems


@functools.partial(pl.kernel, out_type=_agg_out, mesh=_mesh,
                   compiler_params=_sc_params, scratch_types=_agg_scratch)
def _agg_kernel(*args):
    _agg_body(False, args)


@functools.partial(pl.kernel, out_type=_agg_out, mesh=_mesh,
                   compiler_params=_sc_params, scratch_types=_agg_scale_scratch)
def _agg_scale_kernel(*args):
    _agg_body(True, args)


BR = 1000


def _pre_body(x_ref, wm1_ref, wm2_ref, cm_ref, bm2_ref, weab_ref,
              nm_ref, xm_ref, ab_ref):
    x = x_ref[...]
    hmid = jnp.maximum(
        jnp.dot(x, wm1_ref[...], preferred_element_type=jnp.float32)
        + cm_ref[...], 0.0)
    nm = jax.nn.sigmoid(
        jnp.dot(hmid, wm2_ref[...], preferred_element_type=jnp.float32)
        + bm2_ref[...])
    nm_ref[...] = nm
    xm_ref[...] = x * nm
    ab_ref[...] = jnp.dot(x, weab_ref[...],
                          preferred_element_type=jnp.float32)


def _pre_call(x, wm1a, wm2, cm, bm2, weab):
    grid = (N // BR,)
    full = lambda shape: pl.BlockSpec(shape, lambda i: (0, 0))
    return pl.pallas_call(
        _pre_body,
        grid=grid,
        in_specs=[
            pl.BlockSpec((BR, D), lambda i: (i, 0)),
            full((D, MH)), full((MH, 1)), full((1, MH)), full((1, 1)),
            full((D, 2 * MH)),
        ],
        out_specs=[
            pl.BlockSpec((BR, 1), lambda i: (i, 0)),
            pl.BlockSpec((BR, D), lambda i: (i, 0)),
            pl.BlockSpec((BR, 2 * MH), lambda i: (i, 0)),
        ],
        out_shape=[
            jax.ShapeDtypeStruct((N, 1), jnp.float32),
            jax.ShapeDtypeStruct((N, D), jnp.float32),
            jax.ShapeDtypeStruct((N, 2 * MH), jnp.float32),
        ],
    )(x, wm1a, wm2, cm, bm2, weab)


def _conv_body(inp_ref, p0_ref, p1_ref, dinv_ref, ws_ref, bs_ref, wn_ref,
               bn_ref, out_ref):
    agg = (p0_ref[...] + p1_ref[...]) * dinv_ref[...]
    out_ref[...] = jnp.maximum(
        jnp.dot(inp_ref[...], ws_ref[...], preferred_element_type=jnp.float32)
        + bs_ref[...]
        + jnp.dot(agg, wn_ref[...], preferred_element_type=jnp.float32)
        + bn_ref[...], 0.0)


def _conv_call(inp, p0, p1, dinv, ws, bs, wn, bn):
    grid = (N // BR,)
    blk = pl.BlockSpec((BR, H), lambda i: (i, 0))
    full = lambda shape: pl.BlockSpec(shape, lambda i: (0, 0))
    return pl.pallas_call(
        _conv_body,
        grid=grid,
        in_specs=[blk, blk, blk, pl.BlockSpec((BR, 1), lambda i: (i, 0)),
                  full((H, H)), full((1, H)), full((H, H)), full((1, H))],
        out_specs=blk,
        out_shape=jax.ShapeDtypeStruct((N, H), jnp.float32),
    )(inp, p0, p1, dinv, ws, bs, wn, bn)


def _post_body(p0_ref, p1_ref, dinv_ref, h2_ref, sp_ref, pooled_ref):
    i = pl.program_id(0)
    sp = (p0_ref[...] + p1_ref[...]) * dinv_ref[...]
    sp_ref[...] = sp
    contrib = jnp.sum(h2_ref[...] * sp, axis=0, keepdims=True) * (1.0 / N)

    @pl.when(i == 0)
    def _():
        pooled_ref[...] = contrib

    @pl.when(i > 0)
    def _():
        pooled_ref[...] = pooled_ref[...] + contrib


def _post_call(p0, p1, dinv, h2):
    grid = (N // BR,)
    blk = pl.BlockSpec((BR, H), lambda i: (i, 0))
    return pl.pallas_call(
        _post_body,
        grid=grid,
        in_specs=[blk, blk, pl.BlockSpec((BR, 1), lambda i: (i, 0)), blk],
        out_specs=[blk, pl.BlockSpec((1, H), lambda i: (0, 0))],
        out_shape=[jax.ShapeDtypeStruct((N, H), jnp.float32),
                   jax.ShapeDtypeStruct((1, H), jnp.float32)],
    )(p0, p1, dinv, h2)


def _head_body(pooled_ref, gf_ref, wg_ref, bg_ref, wih_ref, bih_ref, whh_ref,
               bhh_ref, wa_ref, ba_ref, wt_ref, bt_ref, wr1_ref, br1_ref,
               wr2_ref, br2_ref, pred_ref, wts_ref, tmask_ref):
    gs = jnp.maximum(
        jnp.dot(gf_ref[...], wg_ref[...], preferred_element_type=jnp.float32)
        + bg_ref[...], 0.0)
    seq = jnp.concatenate([pooled_ref[...], gs], axis=1)
    gi = jnp.dot(seq, wih_ref[...], preferred_element_type=jnp.float32) \
        + bih_ref[...]
    h = jnp.zeros((1, H), jnp.float32)
    outs = []
    for t in range(T):
        gh = jnp.dot(h, whh_ref[...], preferred_element_type=jnp.float32) \
            + bhh_ref[...]
        git = gi[t:t + 1]
        r = jax.nn.sigmoid(git[:, :H] + gh[:, :H])
        z = jax.nn.sigmoid(git[:, H:2 * H] + gh[:, H:2 * H])
        n = jnp.tanh(git[:, 2 * H:] + r * gh[:, 2 * H:])
        h = (1.0 - z) * n + z * h
        outs.append(h)
    gru = jnp.concatenate(outs, axis=0)
    tm = jax.nn.sigmoid(jnp.dot(gru, wt_ref[...],
                                preferred_element_type=jnp.float32)
                        + bt_ref[...])
    sc = jnp.dot(gru, wa_ref[...], preferred_element_type=jnp.float32) \
        + ba_ref[...]
    m = jnp.max(sc)
    ex = jnp.exp(sc - m)
    wts = ex / jnp.sum(ex)
    tmask_ref[...] = tm
    wts_ref[...] = wts
    ctx = jnp.sum(gru * wts * tm, axis=0, keepdims=True)
    pred_ref[...] = jnp.dot(
        jnp.maximum(jnp.dot(ctx, wr1_ref[...],
                            preferred_element_type=jnp.float32)
                    + br1_ref[...], 0.0),
        wr2_ref[...], preferred_element_type=jnp.float32) + br2_ref[...]


def _head_call(pooled3, gf_all, p):
    args = (pooled3, gf_all, p['Wg'], p['bg'].reshape(1, H),
            p['Wih'], p['bih'].reshape(1, 3 * H),
            p['Whh'], p['bhh'].reshape(1, 3 * H),
            p['Wa'], p['ba'].reshape(1, 1), p['Wt'], p['bt'].reshape(1, 1),
            p['Wr1'], p['br1'].reshape(1, H), p['Wr2'],
            p['br2'].reshape(1, 1))
    return pl.pallas_call(
        _head_body,
        out_shape=[jax.ShapeDtypeStruct((1, 1), jnp.float32),
                   jax.ShapeDtypeStruct((T, 1), jnp.float32),
                   jax.ShapeDtypeStruct((T, 1), jnp.float32)],
    )(*args)


def kernel(x_all, edge_index_all, gf_all, params):
    p = params
    weab = jnp.concatenate([p['We1'][:D], p['We1'][D:2 * D]], axis=1)
    wm1a = p['Wm1'][:D]
    iota2 = jnp.arange(2 * NRD, dtype=jnp.int32)
    pad_i = jnp.zeros((EP - E,), jnp.int32)
    pad_d = jnp.full((EP - E,), N, jnp.int32)

    sp_list, em_list, nm_list, pooled_list = [], [], [], []
    for t in range(T):
        tf = float(t) / float(T)
        x = x_all[t]
        src_pad = jnp.concatenate([edge_index_all[t, 0], pad_i])
        dst_pad = jnp.concatenate([edge_index_all[t, 1], pad_d])
        dst2d = dst_pad.reshape(NW, NCH, CH)
        dst2da = dst_pad.reshape(NW, ANCH, ACH)

        ce = tf * p['We1'][2 * D] + p['be1']
        cm = tf * p['Wm1'][D] + p['bm1']
        consts = jnp.zeros((8, 128), jnp.float32)
        consts = consts.at[0, :MH].set(ce)
        consts = consts.at[1, :MH].set(p['We2'][:, 0])
        consts = consts.at[2].set(p['be2'][0])

        nm2d, xm, AB = _pre_call(x, wm1a, p['Wm2'], cm.reshape(1, MH),
                                 p['bm2'].reshape(1, 1), weab)
        ABp = jnp.concatenate([AB, jnp.zeros((8, 2 * MH), jnp.float32)])
        em_pad, degs = _em_kernel(ABp, src_pad, dst2d, consts, iota2)

        degsum = degs[0] + degs[1]
        deg_u = jnp.maximum(degsum[:NRD].reshape(-1)[:N], 1.0)
        deg_e = jnp.maximum(degsum[NRD:].reshape(-1)[:N], 1.0)
        dinv_u = (1.0 / deg_u).reshape(N, 1)
        dinv_e = (1.0 / deg_e).reshape(N, 1)

        parts1 = _agg_kernel(xm, src_pad, dst2da)
        h1 = _conv_call(xm, parts1[0], parts1[1], dinv_u, p['Ws1'],
                        p['bs1'].reshape(1, H), p['Wn1'],
                        p['bn1'].reshape(1, H))
        parts2 = _agg_kernel(h1, src_pad, dst2da)
        h2 = _conv_call(h1, parts2[0], parts2[1], dinv_u, p['Ws2'],
                        p['bs2'].reshape(1, H), p['Wn2'],
                        p['bn2'].reshape(1, H))
        parts3 = _agg_scale_kernel(h2, src_pad, dst2da, em_pad)
        sp, pooled = _post_call(parts3[0], parts3[1], dinv_e, h2)

        sp_list.append(sp)
        em_list.append(em_pad[:E])
        nm_list.append(nm2d[:, 0])
        pooled_list.append(pooled)

    pooled3 = jnp.concatenate(pooled_list, axis=0)
    pred, wts, tmask = _head_call(pooled3, gf_all, p)
    return (pred[0, 0], wts[:, 0], tmask[:, 0], jnp.stack(sp_list),
            jnp.stack(em_list), jnp.stack(nm_list))
